# baseline (speedup 1.0000x reference)
import jax
import jax.numpy as jnp
from jax.experimental import pallas as pl
from jax.experimental.pallas import tpu as pltpu

_B = 8
_H = 512
_W = 512


def _body(g_ref, p_ref, m_ref, s_ref, c_ref, l_ref):
    i = pl.program_id(0)

    @pl.when(i == 0)
    def _init():
        s_ref[0, 0] = 0.0
        c_ref[0, 0] = 0.0
        l_ref[0, 0] = 0.0

    dx = p_ref[0] - g_ref[0]
    dy = p_ref[1] - g_ref[1]
    dz = p_ref[2] - g_ref[2]
    d2 = dx * dx + dy * dy + dz * dz
    dist = jnp.sqrt(d2)
    mf = (m_ref[0] != 0).astype(jnp.float32)
    s_ref[0, 0] += jnp.sum(dist * mf)
    c_ref[0, 0] += jnp.sum(mf)

    @pl.when(i == pl.num_programs(0) - 1)
    def _fin():
        cnt = c_ref[0, 0]
        tot = s_ref[0, 0]
        l_ref[0, 0] = jnp.where(cnt > 0.0, tot / jnp.maximum(cnt, 1.0), 0.0)


def kernel(gt_pts3d, pred_pts3d, valid_mask):
    gp = jnp.transpose(gt_pts3d, (0, 3, 1, 2)).reshape(3 * _B, _H, _W)
    pp = jnp.transpose(pred_pts3d, (0, 3, 1, 2)).reshape(3 * _B, _H, _W)

    plane_spec = pl.BlockSpec((3, _H // 2, _W), lambda i: (i // 2, i % 2, 0))
    mask_spec = pl.BlockSpec((1, _H // 2, _W), lambda i: (i // 2, i % 2, 0))
    scalar_spec = pl.BlockSpec(memory_space=pltpu.SMEM)
    _, _, l = pl.pallas_call(
        _body,
        grid=(2 * _B,),
        in_specs=[plane_spec, plane_spec, mask_spec],
        out_specs=[scalar_spec, scalar_spec, scalar_spec],
        out_shape=[
            jax.ShapeDtypeStruct((1, 1), jnp.float32),
            jax.ShapeDtypeStruct((1, 1), jnp.float32),
            jax.ShapeDtypeStruct((1, 1), jnp.float32),
        ],
    )(gp, pp, valid_mask.view(jnp.int8))
    return (l[0, 0], valid_mask)


# speedup vs baseline: 1.1501x; 1.1501x over previous
import jax
import jax.numpy as jnp
from jax.experimental import pallas as pl
from jax.experimental.pallas import tpu as pltpu

_B = 8
_H = 512
_W = 512


def _body(g_ref, p_ref, m_ref, s_ref, c_ref, l_ref):
    i = pl.program_id(0)

    @pl.when(i == 0)
    def _init():
        s_ref[0, 0] = 0.0
        c_ref[0, 0] = 0.0
        l_ref[0, 0] = 0.0

    dx = p_ref[0] - g_ref[0]
    dy = p_ref[1] - g_ref[1]
    dz = p_ref[2] - g_ref[2]
    d2 = dx * dx + dy * dy + dz * dz
    dist = jnp.sqrt(d2)
    mf = (m_ref[0] != 0).astype(jnp.float32)
    s_ref[0, 0] += jnp.sum(dist * mf)
    c_ref[0, 0] += jnp.sum(mf)

    @pl.when(i == pl.num_programs(0) - 1)
    def _fin():
        cnt = c_ref[0, 0]
        tot = s_ref[0, 0]
        l_ref[0, 0] = jnp.where(cnt > 0.0, tot / jnp.maximum(cnt, 1.0), 0.0)


def kernel(gt_pts3d, pred_pts3d, valid_mask):
    gp = jnp.transpose(gt_pts3d, (0, 3, 1, 2)).reshape(3 * _B, _H, _W)
    pp = jnp.transpose(pred_pts3d, (0, 3, 1, 2)).reshape(3 * _B, _H, _W)

    plane_spec = pl.BlockSpec((3, _H, _W), lambda i: (i, 0, 0))
    mask_spec = pl.BlockSpec((1, _H, _W), lambda i: (i, 0, 0))
    scalar_spec = pl.BlockSpec(memory_space=pltpu.SMEM)
    _, _, l = pl.pallas_call(
        _body,
        grid=(_B,),
        in_specs=[plane_spec, plane_spec, mask_spec],
        out_specs=[scalar_spec, scalar_spec, scalar_spec],
        out_shape=[
            jax.ShapeDtypeStruct((1, 1), jnp.float32),
            jax.ShapeDtypeStruct((1, 1), jnp.float32),
            jax.ShapeDtypeStruct((1, 1), jnp.float32),
        ],
    )(gp, pp, valid_mask.view(jnp.int8))
    return (l[0, 0], valid_mask)
